# baseline (speedup 1.0000x reference)
import functools

import jax
import jax.numpy as jnp
from jax import lax
from jax.experimental import pallas as pl
from jax.experimental.pallas import tpu as pltpu
from jax.experimental.pallas import tpu_sc as plsc

NUM_FIELDS = 26
VOCAB = 100000
EMBED_DIM = 16
BATCH = 16384

N_JOBS = NUM_FIELDS * EMBED_DIM
NW = 32
JOBS_PER_W = N_JOBS // NW
VECS = BATCH // 16

TLEN = 33280
TAIL0 = 3 * TLEN
TAILN = VOCAB - TAIL0
N_STAGES = 3 * JOBS_PER_W
DUMP = BATCH


def _build_sc_gather():
    mesh = plsc.VectorSubcoreMesh(core_axis_name="c", subcore_axis_name="s")

    @functools.partial(
        pl.kernel,
        out_type=jax.ShapeDtypeStruct((N_JOBS, BATCH), jnp.float32),
        mesh=mesh,
        scratch_types=[
            pltpu.VMEM((BATCH + 16,), jnp.int32),
            pltpu.VMEM((BATCH + 80,), jnp.int32),
            pltpu.VMEM((TLEN,), jnp.float32),
            pltpu.VMEM((TLEN,), jnp.float32),
            pltpu.VMEM((TAILN,), jnp.float32),
            pltpu.VMEM((BATCH + 16,), jnp.float32),
            pltpu.SMEM((4,), jnp.int32),
            pltpu.SemaphoreType.DMA,
            pltpu.SemaphoreType.DMA,
        ],
        compiler_params=pltpu.CompilerParams(use_tc_tiling_on_sc=True,
                                             needs_layout_passes=False,
                                             disable_bounds_checks=True),
    )
    def gather_kernel(xt_hbm, tab_hbm, tail_hbm, out_hbm, col_v, bpos_v,
                      slab_a, slab_b, tail_v, obuf_v, ends_s, s_sem, w_sem):
        wid = lax.axis_index("s") * 2 + lax.axis_index("c")
        j0 = wid * JOBS_PER_W
        iota = lax.iota(jnp.int32, 16)

        def fire(g, slab_ref):
            t = g // 3
            h = g - t * 3
            j = j0 + t
            f = j // EMBED_DIM
            d = j - f * EMBED_DIM
            off = pl.multiple_of(h * TLEN, 128)
            pltpu.async_copy(
                tab_hbm.at[f, d].at[pl.ds(off, TLEN)],
                slab_ref, s_sem)

        def bin_pass(lo, hi, cursor):
            def body(p, c):
                v = col_v[pl.ds(p * 16, 16)]
                m = jnp.logical_and(v >= lo, v < hi)
                pos = p * 16 + iota
                plsc.store_compressed(bpos_v.at[pl.ds(c, 16)], pos, mask=m)
                cnt = plsc.all_reduce_population_count(m)[0]
                return c + cnt

            c = lax.fori_loop(0, VECS, body, cursor)
            plsc.store_compressed(
                bpos_v.at[pl.ds(c, 16)],
                jnp.full((16,), DUMP, jnp.int32),
                mask=jnp.full((16,), True, jnp.bool_))
            return (c + 15) & ~15

        def gather_bin(src_ref, base, start, end):
            col_v[pl.ds(BATCH, 16)] = jnp.full((16,), base, jnp.int32)

            def gbody(p, c2):
                pos = bpos_v[pl.ds(start + p * 16, 16)]
                xval = plsc.load_gather(col_v, [pos])
                vals = plsc.load_gather(src_ref, [xval - base])
                plsc.store_scatter(obuf_v, [pos], vals)
                return c2

            lax.fori_loop(0, (end - start) // 16, gbody, 0)

        def do_job_stage(g, slab_ref, next_slab_ref):
            t = g // 3
            h = g - t * 3
            j = j0 + t
            f = j // EMBED_DIM
            d = j - f * EMBED_DIM

            @pl.when(jnp.logical_and(h == 0,
                                     jnp.logical_or(t == 0, d == 0)))
            def _():
                pltpu.sync_copy(xt_hbm.at[f], col_v.at[pl.ds(0, BATCH)])
                e0 = bin_pass(jnp.int32(-1), jnp.int32(TLEN), 0)
                e1 = bin_pass(jnp.int32(TLEN), jnp.int32(2 * TLEN), e0)
                e2 = bin_pass(jnp.int32(2 * TLEN), jnp.int32(TAIL0), e1)
                e3 = bin_pass(jnp.int32(TAIL0), jnp.int32(VOCAB), e2)
                ends_s[0] = e0
                ends_s[1] = e1
                ends_s[2] = e2
                ends_s[3] = e3

            @pl.when(jnp.logical_and(h == 0, t > 0))
            def _():
                pltpu.make_async_copy(obuf_v.at[pl.ds(0, BATCH)],
                                      out_hbm.at[j0], w_sem).wait()

            pltpu.make_async_copy(
                tab_hbm.at[0, 0].at[pl.ds(0, TLEN)],
                slab_ref, s_sem).wait()

            start = jnp.where(h == 0, 0,
                              jnp.where(h == 1, ends_s[0], ends_s[1]))
            end = jnp.where(h == 0, ends_s[0],
                            jnp.where(h == 1, ends_s[1], ends_s[2]))
            gather_bin(slab_ref, h * TLEN, start, end)

            @pl.when(g + 2 < N_STAGES)
            def _():
                fire(g + 2, slab_ref)

            @pl.when(h == 2)
            def _():
                pltpu.sync_copy(tail_hbm.at[f, d], tail_v)
                gather_bin(tail_v, jnp.int32(TAIL0), ends_s[2], ends_s[3])
                pltpu.async_copy(obuf_v.at[pl.ds(0, BATCH)],
                                 out_hbm.at[j], w_sem)

        fire(0, slab_a)
        fire(1, slab_b)

        def pair_body(q, carry):
            do_job_stage(2 * q, slab_a, slab_b)
            do_job_stage(2 * q + 1, slab_b, slab_a)
            return carry

        lax.fori_loop(0, N_STAGES // 2, pair_body, 0)
        do_job_stage(N_STAGES - 1, slab_a, slab_b)
        pltpu.make_async_copy(obuf_v.at[pl.ds(0, BATCH)],
                              out_hbm.at[j0], w_sem).wait()

    return gather_kernel


_sc_gather = _build_sc_gather()


@jax.jit
def kernel(x, tables):
    xt = x.astype(jnp.int32).T
    tabt = tables.transpose(0, 2, 1)
    tails = tabt[:, :, TAIL0:]
    out = _sc_gather(xt, tabt, tails)
    return out.reshape(NUM_FIELDS, EMBED_DIM, BATCH).transpose(2, 0, 1)


# speedup vs baseline: 1.6908x; 1.6908x over previous
import functools

import jax
import jax.numpy as jnp
from jax import lax
from jax.experimental import pallas as pl
from jax.experimental.pallas import tpu as pltpu
from jax.experimental.pallas import tpu_sc as plsc

NUM_FIELDS = 26
VOCAB = 100000
EMBED_DIM = 16
BATCH = 16384

N_JOBS = NUM_FIELDS * EMBED_DIM
NW = 32
JOBS_PER_W = N_JOBS // NW
CHUNK = 4096
N_CHUNKS = BATCH // CHUNK
UNROLL = 8


def _build_sc_gather():
    mesh = plsc.VectorSubcoreMesh(core_axis_name="c", subcore_axis_name="s")

    @functools.partial(
        pl.kernel,
        out_type=jax.ShapeDtypeStruct((N_JOBS, BATCH), jnp.float32),
        mesh=mesh,
        scratch_types=[
            pltpu.VMEM((VOCAB,), jnp.float32),
            pltpu.VMEM((BATCH,), jnp.int32),
            pltpu.VMEM((2, CHUNK), jnp.float32),
            pltpu.SemaphoreType.DMA,
        ],
        compiler_params=pltpu.CompilerParams(use_tc_tiling_on_sc=True,
                                             needs_layout_passes=False),
    )
    def gather_kernel(xt_hbm, tab_hbm, out_hbm, slab_v, col_v, obuf_v,
                      w_sem):
        wid = lax.axis_index("s") * 2 + lax.axis_index("c")
        j0 = wid * JOBS_PER_W

        def do_job(t, carry):
            j = j0 + t
            f = j // EMBED_DIM
            pltpu.sync_copy(tab_hbm.at[f, j - f * EMBED_DIM], slab_v)

            @pl.when(jnp.logical_or(t == 0, f * EMBED_DIM == j))
            def _():
                pltpu.sync_copy(xt_hbm.at[f], col_v)

            for k in range(N_CHUNKS):
                half = k % 2
                if k >= 2:
                    pltpu.make_async_copy(
                        obuf_v.at[half],
                        out_hbm.at[j0, pl.ds(0, CHUNK)], w_sem).wait()
                elif k < 2:
                    @pl.when(t > 0)
                    def _():
                        pltpu.make_async_copy(
                            obuf_v.at[half],
                            out_hbm.at[j0, pl.ds(0, CHUNK)], w_sem).wait()

                def gath(p, c2):
                    base = k * CHUNK + p * (16 * UNROLL)
                    for u in range(UNROLL):
                        sl = pl.ds(base + u * 16, 16)
                        osl = pl.ds(base + u * 16 - k * CHUNK, 16)
                        idx = col_v[sl]
                        obuf_v[half, osl] = plsc.load_gather(slab_v, [idx])
                    return c2

                lax.fori_loop(0, CHUNK // (16 * UNROLL), gath, 0)
                pltpu.async_copy(obuf_v.at[half],
                                 out_hbm.at[j, pl.ds(k * CHUNK, CHUNK)],
                                 w_sem)
            return carry

        lax.fori_loop(0, JOBS_PER_W, do_job, 0)
        for _ in range(2):
            pltpu.make_async_copy(obuf_v.at[0],
                                  out_hbm.at[j0, pl.ds(0, CHUNK)],
                                  w_sem).wait()

    return gather_kernel


_sc_gather = _build_sc_gather()


@jax.jit
def kernel(x, tables):
    xt = x.astype(jnp.int32).T
    tabt = tables.transpose(0, 2, 1)
    out = _sc_gather(xt, tabt)
    return out.reshape(NUM_FIELDS, EMBED_DIM, BATCH).transpose(2, 0, 1)


# speedup vs baseline: 1.7000x; 1.0055x over previous
import functools

import jax
import jax.numpy as jnp
from jax import lax
from jax.experimental import pallas as pl
from jax.experimental.pallas import tpu as pltpu
from jax.experimental.pallas import tpu_sc as plsc

NUM_FIELDS = 26
VOCAB = 100000
EMBED_DIM = 16
BATCH = 16384

N_JOBS = NUM_FIELDS * EMBED_DIM
NW = 32
JOBS_PER_W = N_JOBS // NW
CHUNK = 4096
N_CHUNKS = BATCH // CHUNK
UNROLL = 32


def _build_sc_gather():
    mesh = plsc.VectorSubcoreMesh(core_axis_name="c", subcore_axis_name="s")

    @functools.partial(
        pl.kernel,
        out_type=jax.ShapeDtypeStruct((N_JOBS, BATCH), jnp.float32),
        mesh=mesh,
        scratch_types=[
            pltpu.VMEM((VOCAB,), jnp.float32),
            pltpu.VMEM((BATCH,), jnp.int32),
            pltpu.VMEM((2, CHUNK), jnp.float32),
            pltpu.SemaphoreType.DMA,
        ],
        compiler_params=pltpu.CompilerParams(use_tc_tiling_on_sc=True,
                                             needs_layout_passes=False),
    )
    def gather_kernel(xt_hbm, tab_hbm, out_hbm, slab_v, col_v, obuf_v,
                      w_sem):
        wid = lax.axis_index("s") * 2 + lax.axis_index("c")
        j0 = wid * JOBS_PER_W

        def do_job(t, carry):
            j = j0 + t
            f = j // EMBED_DIM
            pltpu.sync_copy(tab_hbm.at[f, j - f * EMBED_DIM], slab_v)

            @pl.when(jnp.logical_or(t == 0, f * EMBED_DIM == j))
            def _():
                pltpu.sync_copy(xt_hbm.at[f], col_v)

            for k in range(N_CHUNKS):
                half = k % 2
                if k >= 2:
                    pltpu.make_async_copy(
                        obuf_v.at[half],
                        out_hbm.at[j0, pl.ds(0, CHUNK)], w_sem).wait()
                elif k < 2:
                    @pl.when(t > 0)
                    def _():
                        pltpu.make_async_copy(
                            obuf_v.at[half],
                            out_hbm.at[j0, pl.ds(0, CHUNK)], w_sem).wait()

                def gath(p, c2):
                    base = k * CHUNK + p * (16 * UNROLL)
                    for u in range(UNROLL):
                        sl = pl.ds(base + u * 16, 16)
                        osl = pl.ds(base + u * 16 - k * CHUNK, 16)
                        idx = col_v[sl]
                        obuf_v[half, osl] = plsc.load_gather(slab_v, [idx])
                    return c2

                lax.fori_loop(0, CHUNK // (16 * UNROLL), gath, 0)
                pltpu.async_copy(obuf_v.at[half],
                                 out_hbm.at[j, pl.ds(k * CHUNK, CHUNK)],
                                 w_sem)
            return carry

        lax.fori_loop(0, JOBS_PER_W, do_job, 0)
        for _ in range(2):
            pltpu.make_async_copy(obuf_v.at[0],
                                  out_hbm.at[j0, pl.ds(0, CHUNK)],
                                  w_sem).wait()

    return gather_kernel


_sc_gather = _build_sc_gather()


@jax.jit
def kernel(x, tables):
    xt = x.astype(jnp.int32).T
    tabt = tables.transpose(0, 2, 1)
    out = _sc_gather(xt, tabt)
    return out.reshape(NUM_FIELDS, EMBED_DIM, BATCH).transpose(2, 0, 1)
